# baseline (speedup 1.0000x reference)
import functools
import math

import jax
import jax.numpy as jnp
from jax import lax
from jax.experimental import pallas as pl
from jax.experimental.pallas import tpu as pltpu
from jax.experimental.pallas import tpu_sc as plsc

_NC = 2
_NS = 16
_NW = _NC * _NS
_LANES = 16
_CHUNK = 128


@functools.lru_cache(maxsize=None)
def _make_lookup(B, V, D, scale):
    assert B % (8 * _NW) == 0
    assert D % _LANES == 0
    b_per_w = B // _NW
    assert b_per_w % _CHUNK == 0
    n_chunks = b_per_w // _CHUNK
    mesh = plsc.VectorSubcoreMesh(core_axis_name="c", subcore_axis_name="s")

    @functools.partial(
        pl.kernel,
        mesh=mesh,
        out_type=jax.ShapeDtypeStruct((B, D), jnp.float32),
        scratch_types=[
            pltpu.VMEM((b_per_w,), jnp.int32),
            pltpu.VMEM((_CHUNK, D), jnp.float32),
            pltpu.SemaphoreType.DMA,
        ],
        compiler_params=pltpu.CompilerParams(use_tc_tiling_on_sc=False),
    )
    def lookup(idx_hbm, table_hbm, out_hbm, idx_v, rows_v, sem):
        wid = lax.axis_index("s") * _NC + lax.axis_index("c")
        base = wid * b_per_w
        pltpu.sync_copy(idx_hbm.at[pl.ds(base, b_per_w)], idx_v)

        def chunk_body(ci, carry):
            start = pl.multiple_of(ci * _CHUNK, _CHUNK)
            pltpu.async_copy(
                table_hbm.at[idx_v.at[pl.ds(start, _CHUNK)]], rows_v, sem
            ).wait()

            def row_body(r, c2):
                for j in range(D // _LANES):
                    sl = rows_v[r, pl.ds(j * _LANES, _LANES)]
                    rows_v[r, pl.ds(j * _LANES, _LANES)] = sl * scale
                return c2

            lax.fori_loop(0, _CHUNK, row_body, 0, unroll=4)
            pltpu.sync_copy(rows_v, out_hbm.at[pl.ds(base + start, _CHUNK)])
            return carry

        lax.fori_loop(0, n_chunks, chunk_body, 0)

    return lookup


def kernel(tokens, table):
    n, t = tokens.shape
    V, D = table.shape
    B = n * t
    idx = tokens.reshape(B).astype(jnp.int32)
    out = _make_lookup(B, V, D, float(math.sqrt(D)))(idx, table)
    return out.reshape(n, t, D)
